# baseline (speedup 1.0000x reference)
import jax
import jax.numpy as jnp
from jax.experimental import pallas as pl
from jax.experimental.pallas import tpu as pltpu

_D_MODEL = 4096
_N_EXPERTS = 64
_N_CLASSES = 16
_GROUP = 4
_MARGIN = 0.1
_BOOST_EPS = 0.0001


def _router_kernel(x_ref, wt_ref, b_ref, o_ref):
    x = x_ref[...]
    wt = wt_ref[...]
    logits = jnp.dot(x, wt, preferred_element_type=jnp.float32) + b_ref[...]

    bt = logits.shape[0]
    l3 = logits.reshape(bt, _N_CLASSES, _GROUP)
    mx = jnp.max(l3, axis=-1, keepdims=True)
    within = (mx - l3) < _MARGIN
    cnt = jnp.sum(within.astype(jnp.int32), axis=-1, keepdims=True)
    should = cnt > 1
    is_canon = (
        jax.lax.broadcasted_iota(jnp.int32, (bt, _N_CLASSES, _GROUP), 2) == 0
    )
    out = jnp.where(is_canon & should, mx + _BOOST_EPS, l3)
    o_ref[...] = out.reshape(bt, _N_EXPERTS)


def kernel(hidden_states, W, b):
    T, D = hidden_states.shape
    BT = 512
    wt = W.T
    b2 = b.reshape(1, _N_EXPERTS)
    return pl.pallas_call(
        _router_kernel,
        grid=(T // BT,),
        in_specs=[
            pl.BlockSpec((BT, D), lambda i: (i, 0)),
            pl.BlockSpec((D, _N_EXPERTS), lambda i: (0, 0)),
            pl.BlockSpec((1, _N_EXPERTS), lambda i: (0, 0)),
        ],
        out_specs=pl.BlockSpec((BT, _N_EXPERTS), lambda i: (i, 0)),
        out_shape=jax.ShapeDtypeStruct((T, _N_EXPERTS), jnp.float32),
        compiler_params=pltpu.CompilerParams(
            dimension_semantics=("parallel",),
        ),
    )(hidden_states, wt, b2)


# speedup vs baseline: 1.0097x; 1.0097x over previous
import jax
import jax.numpy as jnp
from jax.experimental import pallas as pl
from jax.experimental.pallas import tpu as pltpu

_D_MODEL = 4096
_N_EXPERTS = 64
_N_CLASSES = 16
_GROUP = 4
_MARGIN = 0.1
_BOOST_EPS = 0.0001


def _router_kernel(x_ref, wt_ref, b_ref, o_ref):
    x = x_ref[...]
    wt = wt_ref[...]
    logits = jnp.dot(x, wt, preferred_element_type=jnp.float32) + b_ref[...]

    bt = logits.shape[0]
    l3 = logits.reshape(bt, _N_CLASSES, _GROUP)
    mx = jnp.max(l3, axis=-1, keepdims=True)
    within = (mx - l3) < _MARGIN
    cnt = jnp.sum(within.astype(jnp.int32), axis=-1, keepdims=True)
    should = cnt > 1
    is_canon = (
        jax.lax.broadcasted_iota(jnp.int32, (bt, _N_CLASSES, _GROUP), 2) == 0
    )
    out = jnp.where(is_canon & should, mx + _BOOST_EPS, l3)
    o_ref[...] = out.reshape(bt, _N_EXPERTS)


def kernel(hidden_states, W, b):
    T, D = hidden_states.shape
    BT = 1024
    wt = W.T
    b2 = b.reshape(1, _N_EXPERTS)
    return pl.pallas_call(
        _router_kernel,
        grid=(T // BT,),
        in_specs=[
            pl.BlockSpec((BT, D), lambda i: (i, 0)),
            pl.BlockSpec((D, _N_EXPERTS), lambda i: (0, 0)),
            pl.BlockSpec((1, _N_EXPERTS), lambda i: (0, 0)),
        ],
        out_specs=pl.BlockSpec((BT, _N_EXPERTS), lambda i: (i, 0)),
        out_shape=jax.ShapeDtypeStruct((T, _N_EXPERTS), jnp.float32),
        compiler_params=pltpu.CompilerParams(
            dimension_semantics=("parallel",),
        ),
    )(hidden_states, wt, b2)


# speedup vs baseline: 2.4523x; 2.4287x over previous
import numpy as np
import jax
import jax.numpy as jnp
from jax.experimental import pallas as pl
from jax.experimental.pallas import tpu as pltpu

_D_MODEL = 4096
_N_EXPERTS = 64
_GROUP = 4
_MARGIN = 0.1
_BOOST_EPS = 0.0001


def _perm_matrix(xor_bit):
    p = np.zeros((_N_EXPERTS, _N_EXPERTS), dtype=np.float32)
    for i in range(_N_EXPERTS):
        p[i, i ^ xor_bit] = 1.0
    return p


def _router_kernel(x_ref, wt_ref, b_ref, p1_ref, p2_ref, o_ref):
    x = x_ref[...]
    logits = jnp.dot(x, wt_ref[...], preferred_element_type=jnp.float32)
    logits = logits + b_ref[...]
    p1 = p1_ref[...]
    p2 = p2_ref[...]

    y = jnp.maximum(logits, jnp.dot(logits, p1, preferred_element_type=jnp.float32))
    mx = jnp.maximum(y, jnp.dot(y, p2, preferred_element_type=jnp.float32))

    w = ((mx - logits) < _MARGIN).astype(jnp.float32)
    c = w + jnp.dot(w, p1, preferred_element_type=jnp.float32)
    cnt = c + jnp.dot(c, p2, preferred_element_type=jnp.float32)

    bt = logits.shape[0]
    lane = jax.lax.broadcasted_iota(jnp.int32, (bt, _N_EXPERTS), 1)
    overwrite = ((lane % _GROUP) == 0) & (cnt > 1.5)
    o_ref[...] = jnp.where(overwrite, mx + _BOOST_EPS, logits)


def kernel(hidden_states, W, b):
    T, D = hidden_states.shape
    BT = 1024
    wt = W.T
    b2 = b.reshape(1, _N_EXPERTS)
    p1 = jnp.asarray(_perm_matrix(1))
    p2 = jnp.asarray(_perm_matrix(2))
    return pl.pallas_call(
        _router_kernel,
        grid=(T // BT,),
        in_specs=[
            pl.BlockSpec((BT, D), lambda i: (i, 0)),
            pl.BlockSpec((D, _N_EXPERTS), lambda i: (0, 0)),
            pl.BlockSpec((1, _N_EXPERTS), lambda i: (0, 0)),
            pl.BlockSpec((_N_EXPERTS, _N_EXPERTS), lambda i: (0, 0)),
            pl.BlockSpec((_N_EXPERTS, _N_EXPERTS), lambda i: (0, 0)),
        ],
        out_specs=pl.BlockSpec((BT, _N_EXPERTS), lambda i: (i, 0)),
        out_shape=jax.ShapeDtypeStruct((T, _N_EXPERTS), jnp.float32),
        compiler_params=pltpu.CompilerParams(
            dimension_semantics=("parallel",),
        ),
    )(hidden_states, wt, b2, p1, p2)
